# Initial kernel scaffold; baseline (speedup 1.0000x reference)
import jax
import jax.numpy as jnp
from jax.experimental import pallas as pl


def kernel(x, edge_index, W1, b1, W2, b2, prelu_a, ln_w, ln_b):
    raise NotImplementedError("write your pallas kernel here")


# speedup vs baseline: 9.1075x; 9.1075x over previous
import functools

import jax
import jax.numpy as jnp
from jax import lax
from jax.experimental import pallas as pl
from jax.experimental.pallas import tpu as pltpu
from jax.experimental.pallas import tpu_sc as plsc

N_NODES = 10000
D = 128
EPS = 1e-5

NC = 2
NS = 16
NW = NC * NS
CHUNK = 128
DEG_W = 128
N_PAD = 10240
ROWS_PT = N_PAD // NS

_MESH = dict(core_axis_name="c", subcore_axis_name="s", num_cores=NC,
             num_subcores=NS)


@functools.lru_cache(maxsize=None)
def _deg_kernel(e_pad):
  ept = e_pad // NW
  nchunks = ept // CHUNK

  @functools.partial(
      pl.kernel,
      out_type=jax.ShapeDtypeStruct((NC, N_PAD, DEG_W), jnp.float32),
      mesh=plsc.VectorSubcoreMesh(**_MESH),
      scratch_types=[
          pltpu.VMEM((CHUNK,), jnp.int32),
          pltpu.VMEM((CHUNK, DEG_W), jnp.float32),
          pltpu.VMEM_SHARED((N_PAD, DEG_W), jnp.float32),
      ],
  )
  def deg(dst_hbm, ones_hbm, zeros_hbm, out_hbm, idx_v, ones_v, acc_sh):
    cid = lax.axis_index("c")
    sid = lax.axis_index("s")
    pltpu.sync_copy(zeros_hbm, acc_sh.at[pl.ds(sid * ROWS_PT, ROWS_PT)])
    pltpu.sync_copy(ones_hbm, ones_v)
    plsc.subcore_barrier()
    base = (cid * NS + sid) * ept

    def step(i, carry):
      pltpu.sync_copy(dst_hbm.at[pl.ds(base + i * CHUNK, CHUNK)], idx_v)
      pltpu.sync_copy(ones_v, acc_sh.at[idx_v], add=True)
      return carry

    lax.fori_loop(0, nchunks, step, 0)
    plsc.subcore_barrier()
    pltpu.sync_copy(acc_sh.at[pl.ds(sid * ROWS_PT, ROWS_PT)],
                    out_hbm.at[cid, pl.ds(sid * ROWS_PT, ROWS_PT)])

  return deg


@functools.lru_cache(maxsize=None)
def _scatter_kernel(e_pad):
  ept = e_pad // NW
  nchunks = ept // CHUNK

  @functools.partial(
      pl.kernel,
      out_type=jax.ShapeDtypeStruct((NC, N_PAD, D), jnp.float32),
      mesh=plsc.VectorSubcoreMesh(**_MESH),
      scratch_types=[
          pltpu.VMEM((CHUNK,), jnp.int32),
          pltpu.VMEM((CHUNK,), jnp.int32),
          pltpu.VMEM((CHUNK, D), jnp.float32),
          pltpu.VMEM_SHARED((N_PAD, D), jnp.float32),
          pltpu.SemaphoreType.DMA,
      ],
  )
  def scat(g_hbm, src_hbm, dst_hbm, zeros_hbm, out_hbm,
           sidx_v, didx_v, rows_v, acc_sh, sem):
    cid = lax.axis_index("c")
    sid = lax.axis_index("s")
    pltpu.sync_copy(zeros_hbm, acc_sh.at[pl.ds(sid * ROWS_PT, ROWS_PT)])
    plsc.subcore_barrier()
    base = (cid * NS + sid) * ept

    def step(i, carry):
      off = base + i * CHUNK
      pltpu.sync_copy(src_hbm.at[pl.ds(off, CHUNK)], sidx_v)
      pltpu.sync_copy(dst_hbm.at[pl.ds(off, CHUNK)], didx_v)
      pltpu.async_copy(g_hbm.at[sidx_v], rows_v, sem).wait()
      pltpu.sync_copy(rows_v, acc_sh.at[didx_v], add=True)
      return carry

    lax.fori_loop(0, nchunks, step, 0)
    plsc.subcore_barrier()
    pltpu.sync_copy(acc_sh.at[pl.ds(sid * ROWS_PT, ROWS_PT)],
                    out_hbm.at[cid, pl.ds(sid * ROWS_PT, ROWS_PT)])

  return scat


BR = 256


def _dinv_block(d0_ref, d1_ref):
  deg = d0_ref[...][:, 0:1] + d1_ref[...][:, 0:1] + 1.0
  return lax.rsqrt(deg)


def _b1_body(x_ref, w_ref, d0_ref, d1_ref, g_ref):
  dinv = _dinv_block(d0_ref, d1_ref)
  g = jnp.dot(x_ref[...], w_ref[...], preferred_element_type=jnp.float32)
  g_ref[...] = g * dinv


def _post_block(p0_ref, p1_ref, g_ref, dinv, b_ref, lw_ref, lb_ref, a_ref):
  h = (p0_ref[...] + p1_ref[...] + g_ref[...]) * dinv + b_ref[...]
  h = jnp.where(h >= 0, h, a_ref[...] * h)
  mu = jnp.mean(h, axis=-1, keepdims=True)
  var = jnp.mean((h - mu) ** 2, axis=-1, keepdims=True)
  return (h - mu) * lax.rsqrt(var + EPS) * lw_ref[...] + lb_ref[...]


def _b2_body(p0_ref, p1_ref, g1_ref, d0_ref, d1_ref, b_ref, lw_ref, lb_ref,
             a_ref, w_ref, out_ref):
  dinv = _dinv_block(d0_ref, d1_ref)
  h = _post_block(p0_ref, p1_ref, g1_ref, dinv, b_ref, lw_ref, lb_ref, a_ref)
  out_ref[...] = jnp.dot(h, w_ref[...],
                         preferred_element_type=jnp.float32) * dinv


def _b3_body(p0_ref, p1_ref, g2_ref, d0_ref, d1_ref, b_ref, lw_ref, lb_ref,
             a_ref, out_ref):
  dinv = _dinv_block(d0_ref, d1_ref)
  out_ref[...] = _post_block(p0_ref, p1_ref, g2_ref, dinv, b_ref, lw_ref,
                             lb_ref, a_ref)


def _row_spec():
  return pl.BlockSpec((BR, D), lambda i: (i, 0))


def _deg_spec():
  return pl.BlockSpec((BR, DEG_W), lambda i: (i, 0))


def _vec_spec():
  return pl.BlockSpec((1, D), lambda i: (0, 0))


def _full_spec():
  return pl.BlockSpec((D, D), lambda i: (0, 0))


_GRID = (N_PAD // BR,)
_ROWS_OUT = jax.ShapeDtypeStruct((N_PAD, D), jnp.float32)


def kernel(x, edge_index, W1, b1, W2, b2, prelu_a, ln_w, ln_b):
  e = edge_index.shape[1]
  ept = -(-e // (NW * CHUNK)) * CHUNK
  e_pad = ept * NW
  pad = jnp.full((e_pad - e,), N_NODES, dtype=jnp.int32)
  src = jnp.concatenate([edge_index[0].astype(jnp.int32), pad])
  dst = jnp.concatenate([edge_index[1].astype(jnp.int32), pad])
  x_pad = jnp.pad(x, ((0, N_PAD - N_NODES), (0, 0)))

  ones_c = jnp.ones((CHUNK, DEG_W), jnp.float32)
  zeros_d = jnp.zeros((ROWS_PT, DEG_W), jnp.float32)
  zeros_r = jnp.zeros((ROWS_PT, D), jnp.float32)

  b1v = b1.reshape(1, D)
  b2v = b2.reshape(1, D)
  lwv = ln_w.reshape(1, D)
  lbv = ln_b.reshape(1, D)
  av = jnp.broadcast_to(prelu_a.reshape(1, 1), (1, D))

  degp = _deg_kernel(e_pad)(dst, ones_c, zeros_d)
  d0, d1 = degp[0], degp[1]

  g1 = pl.pallas_call(
      _b1_body,
      grid=_GRID,
      in_specs=[_row_spec(), _full_spec(), _deg_spec(), _deg_spec()],
      out_specs=_row_spec(),
      out_shape=_ROWS_OUT,
  )(x_pad, W1, d0, d1)

  p = _scatter_kernel(e_pad)(g1, src, dst, zeros_r)

  g2 = pl.pallas_call(
      _b2_body,
      grid=_GRID,
      in_specs=[_row_spec(), _row_spec(), _row_spec(), _deg_spec(),
                _deg_spec(), _vec_spec(), _vec_spec(), _vec_spec(),
                _vec_spec(), _full_spec()],
      out_specs=_row_spec(),
      out_shape=_ROWS_OUT,
  )(p[0], p[1], g1, d0, d1, b1v, lwv, lbv, av, W2)

  p2 = _scatter_kernel(e_pad)(g2, src, dst, zeros_r)

  out = pl.pallas_call(
      _b3_body,
      grid=_GRID,
      in_specs=[_row_spec(), _row_spec(), _row_spec(), _deg_spec(),
                _deg_spec(), _vec_spec(), _vec_spec(), _vec_spec(),
                _vec_spec()],
      out_specs=_row_spec(),
      out_shape=_ROWS_OUT,
  )(p2[0], p2[1], g2, d0, d1, b2v, lwv, lbv, av)

  return out[:N_NODES]
